# baseline (speedup 1.0000x reference)
import functools

import numpy as np
import jax
import jax.numpy as jnp
from jax import lax
from jax.experimental import pallas as pl
from jax.experimental.pallas import tpu as pltpu
from jax.experimental.pallas import tpu_sc as plsc

_D = 64
_L = 200
_B = 1024
_NC = 2
_NS = 16
_NW = _NC * _NS


def _pos_encoding(length, depth):
    positions = np.arange(length).reshape(-1, 1)
    depths = np.array([2 * (i // 2) for i in range(depth)]).reshape(1, -1)
    angle_rates = 1.0 / 10000 ** (depths / depth)
    angles = positions * angle_rates
    encoding = np.cos(angles)
    encoding[:, ::2] = np.sin(encoding[:, ::2])
    return encoding.astype(np.float32)


_ENC = jnp.asarray(_pos_encoding(_L, _D))

_EPW = _B // _NW


_mesh = plsc.VectorSubcoreMesh(core_axis_name="c", subcore_axis_name="s")


@functools.partial(
    pl.kernel,
    mesh=_mesh,
    out_type=jax.ShapeDtypeStruct((_B * _L, _D), jnp.float32),
    scratch_types=[
        pltpu.VMEM((_L, _D), jnp.float32),
        pltpu.VMEM((104,), jnp.int32),
        pltpu.VMEM((96,), jnp.int32),
        pltpu.VMEM((_L, _D), jnp.float32),
        pltpu.SemaphoreType.DMA,
    ],
    compiler_params=pltpu.CompilerParams(use_tc_tiling_on_sc=False),
)
def _emb_kernel(table_hbm, xf_hbm, enc_hbm, out_hbm, enc_v, idx_a, idx_b, rows_v, sem):
    wid = lax.axis_index("s") * _NC + lax.axis_index("c")
    pltpu.sync_copy(enc_hbm, enc_v)

    def elem(e, carry):
        r0 = (wid * _EPW + e) * _L
        pltpu.sync_copy(xf_hbm.at[pl.ds(r0, 104)], idx_a)
        pltpu.sync_copy(xf_hbm.at[pl.ds(r0 + 104, 96)], idx_b)
        cp1 = pltpu.async_copy(table_hbm.at[idx_a], rows_v.at[pl.ds(0, 104)], sem)
        cp2 = pltpu.async_copy(table_hbm.at[idx_b], rows_v.at[pl.ds(104, 96)], sem)
        cp1.wait()
        cp2.wait()

        def row(l, c):
            for j in range(4):
                sl = pl.ds(j * 16, 16)
                rows_v[l, sl] = rows_v[l, sl] + enc_v[l, sl]
            return c

        lax.fori_loop(0, _L, row, 0)
        pltpu.sync_copy(rows_v, out_hbm.at[pl.ds(r0, _L)])
        return carry

    lax.fori_loop(0, _EPW, elem, 0)


@jax.jit
def kernel(x, table):
    xf = x.reshape(-1).astype(jnp.int32)
    out = _emb_kernel(table, xf, _ENC)
    return out.reshape(_B, _L, _D)
